# baseline (speedup 1.0000x reference)
import functools

import jax
import jax.numpy as jnp
from jax import lax
from jax.experimental import pallas as pl
from jax.experimental.pallas import tpu as pltpu
from jax.experimental.pallas import tpu_sc as plsc

G = 4
B = 2500
L = 16
D = 256
A = 512
BPAD = 2560
N = G * BPAD
NC, NS = 2, 16
NW = NC * NS
TILE_ROWS = N // NW
CHUNKS = (128, 128, 64)
NBUF = 8
CH = TILE_ROWS // 4
IDX_PER_TILE = (2 * L + 1) * TILE_ROWS
BM = 512
NBLK = N // BM

@functools.cache
def _sc_kernels():
    mesh = plsc.VectorSubcoreMesh(core_axis_name="c", subcore_axis_name="s",
                                  num_cores=NC, num_subcores=NS)

    @functools.partial(
        pl.kernel,
        out_type=(
            jax.ShapeDtypeStruct((L + 1, N, D // 2), jnp.int32),
            jax.ShapeDtypeStruct((L, N, D // 2), jnp.int32),
        ),
        mesh=mesh,
        scratch_types=[
            pltpu.VMEM((IDX_PER_TILE,), jnp.int32),
            pltpu.VMEM((NBUF, CH, D // 2), jnp.int32),
            pltpu.SemaphoreType.DMA((NBUF,)),
            pltpu.SemaphoreType.DMA((NBUF,)),
        ],
    )
    def sc_gather(idx_hbm, tdx_hbm, tan_hbm,
                  leaf_out, comb_out, idx_v, rowbuf, gsem, wsem):
        wid = lax.axis_index("s") * NC + lax.axis_index("c")
        tile_base = wid * TILE_ROWS
        ibase = pl.multiple_of(wid * IDX_PER_TILE, 64)
        pltpu.sync_copy(idx_hbm.at[pl.ds(ibase, IDX_PER_TILE)], idx_v)

        def fire(table, off, b):
            o = pl.multiple_of(off, 8)
            return pltpu.async_copy(
                table.at[idx_v.at[pl.ds(o, CH)]], rowbuf.at[b], gsem.at[b])

        def drain(dst, l, b, gd):
            gd.wait()
            return pltpu.async_copy(
                rowbuf.at[b], dst.at[l, pl.ds(tile_base + (b % 4) * CH, CH)],
                wsem.at[b])

        @pl.loop(0, L)
        def _(l):
            off = l * (2 * TILE_ROWS)
            g = [fire(tdx_hbm, off + j * CH, j) for j in range(4)]
            g += [fire(tan_hbm, off + TILE_ROWS + j * CH, 4 + j)
                  for j in range(4)]
            w = [drain(leaf_out, l, j, g[j]) for j in range(4)]
            w += [drain(comb_out, l, 4 + j, g[4 + j]) for j in range(4)]
            for wd in w:
                wd.wait()

        g = [fire(tdx_hbm, 2 * L * TILE_ROWS + j * CH, j) for j in range(4)]
        w = [drain(leaf_out, L, j, g[j]) for j in range(4)]
        for wd in w:
            wd.wait()

    @functools.partial(
        pl.kernel,
        out_type=jax.ShapeDtypeStruct((N, D), jnp.float32),
        mesh=mesh,
        scratch_types=[
            pltpu.VMEM((max(CHUNKS),), jnp.int32),
            pltpu.VMEM((max(CHUNKS), D), jnp.float32),
            pltpu.SemaphoreType.DMA,
        ],
    )
    def sc_permute(idx_hbm, emb_hbm, out_hbm, idx_v, rows_v, sem):
        wid = lax.axis_index("s") * NC + lax.axis_index("c")
        off = 0
        for cn in CHUNKS:
            base = wid * TILE_ROWS + off
            pltpu.sync_copy(idx_hbm.at[pl.ds(base, cn)], idx_v.at[pl.ds(0, cn)])
            pltpu.async_copy(emb_hbm.at[idx_v.at[pl.ds(0, cn)]],
                             rows_v.at[pl.ds(0, cn)], sem).wait()
            pltpu.sync_copy(rows_v.at[pl.ds(0, cn)], out_hbm.at[pl.ds(base, cn)])
            off += cn

    return sc_gather, sc_permute


NRPAD = 128
D2 = D // 2


def _tc_attend(leaf_ref, comb_ref, rel_ref, tre_ref, w1_ref, w2_ref, w12_ref,
               b_ref, cw_ref, t_ref, out_ref):
    pres = []
    combs = []
    rel_iota = lax.broadcasted_iota(jnp.int32, (BM, NRPAD), 1)
    tre_bf = tre_ref[...].astype(jnp.bfloat16)
    tw = jnp.dot(tre_bf, w2_ref[...],
                 preferred_element_type=jnp.float32).astype(jnp.bfloat16)
    tb = jnp.dot(t_ref[...].astype(jnp.bfloat16), w2_ref[...],
                 preferred_element_type=jnp.float32)
    for l in range(L + 1):
        lf = leaf_ref[l]
        if l < L:
            oh = (rel_ref[l][:, None] == rel_iota).astype(jnp.float32)
            cb = comb_ref[l].astype(jnp.float32) + jnp.dot(
                oh, tre_ref[...], preferred_element_type=jnp.float32)
            x = jnp.dot(lf, w1_ref[...], preferred_element_type=jnp.float32)
            x = x + jnp.dot(comb_ref[l], w2_ref[...],
                            preferred_element_type=jnp.float32)
            x = x + jnp.dot(oh.astype(jnp.bfloat16), tw,
                            preferred_element_type=jnp.float32)
        else:
            cb = lf.astype(jnp.float32) + t_ref[...]
            x = jnp.dot(lf, w12_ref[...],
                        preferred_element_type=jnp.float32) + tb
        combs.append(cb)
        x = jnp.tanh(x + b_ref[...])
        pres.append(jnp.sum(x * cw_ref[...], axis=1, keepdims=True))
    p = jnp.concatenate(pres, axis=1)
    m = jnp.max(p, axis=1, keepdims=True)
    e = jnp.exp(p - m)
    s = jnp.sum(e, axis=1, keepdims=True)
    acc = combs[0] * (e[:, 0:1] / s)
    for l in range(1, L + 1):
        acc = acc + combs[l] * (e[:, l:l + 1] / s)
    out_ref[...] = acc


def kernel(dxEmb, leavesList, ancestorsList, relationList, permute_index,
           table_dx, table_t, table_an, table_re, attn_w, attn_b, comb_w,
           comb_b):
    del dxEmb, comb_b
    def prep(idx):
        idx = jnp.pad(idx.astype(jnp.int32), ((0, 0), (0, BPAD - B), (0, 0)))
        return idx.transpose(2, 0, 1).reshape(L, N)

    il16 = prep(leavesList)
    il3 = jnp.concatenate([il16, il16[0:1]], axis=0).reshape(L + 1, NW,
                                                             TILE_ROWS)
    ia3 = prep(ancestorsList).reshape(L, NW, TILE_ROWS)
    ib = prep(relationList)
    pairs = jnp.stack([il3[:L], ia3], axis=1)
    head = pairs.transpose(2, 0, 1, 3).reshape(NW, L * 2 * TILE_ROWS)
    all_idx = jnp.concatenate([head, il3[L]], axis=1).reshape(-1)

    def as_i32_rows(t):
        tb = t.astype(jnp.bfloat16).reshape(t.shape[0], D // 2, 2)
        return lax.bitcast_convert_type(tb, jnp.int32)

    def as_bf16_rows(t32):
        tb = lax.bitcast_convert_type(t32, jnp.bfloat16)
        return tb.reshape(*t32.shape[:-1], D)

    sc_gather, sc_permute = _sc_kernels()
    leaf32, comb32 = sc_gather(all_idx, as_i32_rows(table_dx),
                               as_i32_rows(table_an))
    leaf_buf = as_bf16_rows(leaf32)
    comb_buf = as_bf16_rows(comb32)
    tre_pad = jnp.pad(table_re, ((0, NRPAD - (table_re.shape[0])), (0, 0)))

    w1 = attn_w[:D].astype(jnp.bfloat16)
    w2 = attn_w[D:].astype(jnp.bfloat16)
    w12 = (attn_w[:D] + attn_w[D:]).astype(jnp.bfloat16)
    out_full = pl.pallas_call(
        _tc_attend,
        grid=(NBLK,),
        in_specs=[
            pl.BlockSpec((L + 1, BM, D), lambda i: (0, i, 0)),
            pl.BlockSpec((L, BM, D), lambda i: (0, i, 0)),
            pl.BlockSpec((L, BM), lambda i: (0, i)),
            pl.BlockSpec((NRPAD, D), lambda i: (0, 0)),
            pl.BlockSpec((D, A), lambda i: (0, 0)),
            pl.BlockSpec((D, A), lambda i: (0, 0)),
            pl.BlockSpec((D, A), lambda i: (0, 0)),
            pl.BlockSpec((1, A), lambda i: (0, 0)),
            pl.BlockSpec((1, A), lambda i: (0, 0)),
            pl.BlockSpec((1, D), lambda i: (0, 0)),
        ],
        out_specs=pl.BlockSpec((BM, D), lambda i: (i, 0)),
        out_shape=jax.ShapeDtypeStruct((N, D), jnp.float32),
    )(leaf_buf, comb_buf, ib, tre_pad, w1, w2, w12, attn_b.reshape(1, A),
      comb_w.reshape(1, A), table_t)

    allEmb_p = jnp.concatenate(
        [out_full, jnp.zeros((8, D), jnp.float32)], axis=0)
    p = permute_index.astype(jnp.int32)
    mapped = jnp.where(p == G * B, N, (p // B) * BPAD + p % B)
    mapped = jnp.concatenate(
        [mapped, jnp.zeros((N - (G * B + 1),), jnp.int32)])
    out = sc_permute(mapped, allEmb_p)
    return out[:G * B + 1]


# speedup vs baseline: 2.7897x; 2.7897x over previous
import functools

import jax
import jax.numpy as jnp
from jax import lax
from jax.experimental import pallas as pl
from jax.experimental.pallas import tpu as pltpu
from jax.experimental.pallas import tpu_sc as plsc

G = 4
B = 2500
L = 16
D = 256
A = 512
BPAD = 2560
N = G * BPAD
NC, NS = 2, 16
NW = NC * NS
TILE_ROWS = N // NW
CHUNKS = (128, 128, 64)
NBUF = 7
NCH = 5
CH = TILE_ROWS // NCH
IDX_PER_TILE = 2 * L * TILE_ROWS
BM = 512
NBLK = N // BM

@functools.cache
def _sc_kernels():
    mesh = plsc.VectorSubcoreMesh(core_axis_name="c", subcore_axis_name="s",
                                  num_cores=NC, num_subcores=NS)

    @functools.partial(
        pl.kernel,
        out_type=(
            jax.ShapeDtypeStruct((L, N, D), jnp.float32),
            jax.ShapeDtypeStruct((L, N, D), jnp.float32),
        ),
        mesh=mesh,
        scratch_types=[
            pltpu.VMEM((IDX_PER_TILE,), jnp.int32),
            pltpu.VMEM((NBUF, CH, D), jnp.float32),
            pltpu.SemaphoreType.DMA((NBUF,)),
            pltpu.SemaphoreType.DMA((NBUF,)),
        ],
    )
    def sc_gather(idx_hbm, tdx_hbm, tan_hbm,
                  leaf_out, comb_out, idx_v, rowbuf, gsem, wsem):
        wid = lax.axis_index("s") * NC + lax.axis_index("c")
        tile_base = wid * TILE_ROWS
        ibase = pl.multiple_of(wid * IDX_PER_TILE, 64)
        pltpu.sync_copy(idx_hbm.at[pl.ds(ibase, IDX_PER_TILE)], idx_v)

        @pl.loop(0, L)
        def _(l):
            off = l * (2 * TILE_ROWS)
            units = ([(tdx_hbm, leaf_out, c) for c in range(NCH)]
                     + [(tan_hbm, comb_out, c) for c in range(NCH)])
            w = [None] * NBUF
            prev = None
            for u, (table, dst, c) in enumerate(units):
                b = u % NBUF
                if w[b] is not None:
                    w[b].wait()
                o = pl.multiple_of(off + u * CH, 8)
                gd = pltpu.async_copy(
                    table.at[idx_v.at[pl.ds(o, CH)]], rowbuf.at[b], gsem.at[b])
                if prev is not None:
                    pb, pd, pdst, pc = prev
                    pd.wait()
                    w[pb] = pltpu.async_copy(
                        rowbuf.at[pb],
                        pdst.at[l, pl.ds(tile_base + pc * CH, CH)],
                        wsem.at[pb])
                prev = (b, gd, dst, c)
            pb, pd, pdst, pc = prev
            pd.wait()
            w[pb] = pltpu.async_copy(
                rowbuf.at[pb], pdst.at[l, pl.ds(tile_base + pc * CH, CH)],
                wsem.at[pb])
            for wd in w:
                if wd is not None:
                    wd.wait()

    @functools.partial(
        pl.kernel,
        out_type=jax.ShapeDtypeStruct((N, D), jnp.float32),
        mesh=mesh,
        scratch_types=[
            pltpu.VMEM((max(CHUNKS),), jnp.int32),
            pltpu.VMEM((max(CHUNKS), D), jnp.float32),
            pltpu.SemaphoreType.DMA,
        ],
    )
    def sc_permute(idx_hbm, emb_hbm, out_hbm, idx_v, rows_v, sem):
        wid = lax.axis_index("s") * NC + lax.axis_index("c")
        off = 0
        for cn in CHUNKS:
            base = wid * TILE_ROWS + off
            pltpu.sync_copy(idx_hbm.at[pl.ds(base, cn)], idx_v.at[pl.ds(0, cn)])
            pltpu.async_copy(emb_hbm.at[idx_v.at[pl.ds(0, cn)]],
                             rows_v.at[pl.ds(0, cn)], sem).wait()
            pltpu.sync_copy(rows_v.at[pl.ds(0, cn)], out_hbm.at[pl.ds(base, cn)])
            off += cn

    return sc_gather, sc_permute


NRPAD = 128


def _tc_attend(leaf_ref, comb_ref, rel_ref, tre_ref, w1_ref, w2_ref, w12_ref,
               b_ref, cw_ref, t_ref, out_ref):
    pres = []
    combs = []
    rel_iota = lax.broadcasted_iota(jnp.int32, (BM, NRPAD), 1)
    tb = jnp.dot(t_ref[...], w2_ref[...],
                 preferred_element_type=jnp.float32)
    for l in range(L + 1):
        if l < L:
            lf = leaf_ref[l]
            oh = (rel_ref[l][:, None] == rel_iota).astype(jnp.float32)
            cb = comb_ref[l] + jnp.dot(oh, tre_ref[...],
                                       preferred_element_type=jnp.float32)
            x = jnp.dot(lf, w1_ref[...], preferred_element_type=jnp.float32)
            x = x + jnp.dot(cb, w2_ref[...],
                            preferred_element_type=jnp.float32)
        else:
            lf = leaf_ref[0]
            cb = lf + t_ref[...]
            x = jnp.dot(lf, w12_ref[...],
                        preferred_element_type=jnp.float32) + tb
        combs.append(cb)
        x = jnp.tanh(x + b_ref[...])
        pres.append(jnp.sum(x * cw_ref[...], axis=1, keepdims=True))
    p = jnp.concatenate(pres, axis=1)
    m = jnp.max(p, axis=1, keepdims=True)
    e = jnp.exp(p - m)
    s = jnp.sum(e, axis=1, keepdims=True)
    acc = combs[0] * (e[:, 0:1] / s)
    for l in range(1, L + 1):
        acc = acc + combs[l] * (e[:, l:l + 1] / s)
    out_ref[...] = acc


def kernel(dxEmb, leavesList, ancestorsList, relationList, permute_index,
           table_dx, table_t, table_an, table_re, attn_w, attn_b, comb_w,
           comb_b):
    del dxEmb, comb_b
    def prep(idx):
        idx = jnp.pad(idx.astype(jnp.int32), ((0, 0), (0, BPAD - B), (0, 0)))
        return idx.transpose(2, 0, 1).reshape(L, N)

    il3 = prep(leavesList).reshape(L, NW, TILE_ROWS)
    ia3 = prep(ancestorsList).reshape(L, NW, TILE_ROWS)
    ib = prep(relationList)
    pairs = jnp.stack([il3, ia3], axis=1)
    all_idx = pairs.transpose(2, 0, 1, 3).reshape(-1)

    sc_gather, sc_permute = _sc_kernels()
    leaf_buf, comb_buf = sc_gather(all_idx, table_dx, table_an)
    tre_pad = jnp.pad(table_re, ((0, NRPAD - (table_re.shape[0])), (0, 0)))

    w1 = attn_w[:D]
    w2 = attn_w[D:]
    w12 = w1 + w2
    out_full = pl.pallas_call(
        _tc_attend,
        grid=(NBLK,),
        in_specs=[
            pl.BlockSpec((L, BM, D), lambda i: (0, i, 0)),
            pl.BlockSpec((L, BM, D), lambda i: (0, i, 0)),
            pl.BlockSpec((L, BM), lambda i: (0, i)),
            pl.BlockSpec((NRPAD, D), lambda i: (0, 0)),
            pl.BlockSpec((D, A), lambda i: (0, 0)),
            pl.BlockSpec((D, A), lambda i: (0, 0)),
            pl.BlockSpec((D, A), lambda i: (0, 0)),
            pl.BlockSpec((1, A), lambda i: (0, 0)),
            pl.BlockSpec((1, A), lambda i: (0, 0)),
            pl.BlockSpec((1, D), lambda i: (0, 0)),
        ],
        out_specs=pl.BlockSpec((BM, D), lambda i: (i, 0)),
        out_shape=jax.ShapeDtypeStruct((N, D), jnp.float32),
    )(leaf_buf, comb_buf, ib, tre_pad, w1, w2, w12, attn_b.reshape(1, A),
      comb_w.reshape(1, A), table_t)

    allEmb_p = jnp.concatenate(
        [out_full, jnp.zeros((8, D), jnp.float32)], axis=0)
    p = permute_index.astype(jnp.int32)
    mapped = jnp.where(p == G * B, N, (p // B) * BPAD + p % B)
    mapped = jnp.concatenate(
        [mapped, jnp.zeros((N - (G * B + 1),), jnp.int32)])
    out = sc_permute(mapped, allEmb_p)
    return out[:G * B + 1]
